# Initial kernel scaffold; baseline (speedup 1.0000x reference)
import jax
import jax.numpy as jnp
from jax.experimental import pallas as pl


def kernel(x_bnd, x_res, x_atm_l1, pos_atm, pos_res, edge_feat_bnd, edge_feat_res, edge_feat_atm, r2a, ligidx, params, edge_index_bnd, edge_index_res, edge_index_atm):
    raise NotImplementedError("write your pallas kernel here")


# speedup vs baseline: 1.0111x; 1.0111x over previous
import functools

import jax
import jax.numpy as jnp
from jax.experimental import pallas as pl
from jax.experimental.pallas import tpu as pltpu

N_ATM = 10000
N_RES = 1000
C = 32
NL_BND = 2
NL_RES = 4
NL_ATM = 4
NOUT = 11


def _seg_softmax(logits, seg, n):
    m = jax.ops.segment_max(logits, seg, num_segments=n)
    m = jnp.where(jnp.isfinite(m), m, 0.0)
    e = jnp.exp(logits - m[seg])
    s = jax.ops.segment_sum(e, seg, num_segments=n)
    return e / (s[seg] + 1e-9)


def _safe_norm(x):
    return jnp.sqrt(jnp.sum(x * x, axis=-1, keepdims=True) + 1e-12)


def _scalar_layer(h0, src, dst, efeat, rlen, Wq, Wk, Wv, We, Ws, n):
    q = h0 @ Wq
    kk = (h0[src] @ Wk) * (efeat @ We)
    logits = jnp.sum(q[dst] * kk, axis=-1) / jnp.sqrt(float(h0.shape[1]))
    a = _seg_softmax(logits, dst, n)
    radial = jnp.exp(-(rlen ** 2) / 100.0)[:, None]
    v = (h0[src] @ Wv) * radial
    agg = jax.ops.segment_sum(a[:, None] * v, dst, num_segments=n)
    out = agg + h0 @ Ws
    nv = _safe_norm(out)
    return out / (nv + 1e-6) * jax.nn.relu(nv)


def _head_body(h0_ref, h1_ref, lig_ref, w0_ref, c1_ref, c1b_ref, c2_ref,
               c2b_ref, w1s_ref, cat_ref, dxyz_ref):
    h0 = h0_ref[...]
    lig = lig_ref[...]
    h0o = jnp.dot(h0, w0_ref[...], preferred_element_type=jnp.float32)
    z = jax.nn.relu(jnp.dot(h0o, c1_ref[...],
                            preferred_element_type=jnp.float32) + c1b_ref[...])
    cat = jnp.dot(z, c2_ref[...], preferred_element_type=jnp.float32) + c2b_ref[...]
    cat_ref[...] = jnp.dot(lig, cat, preferred_element_type=jnp.float32)
    dxyz_ref[...] = jnp.dot(lig, h1_ref[...],
                            preferred_element_type=jnp.float32) * w1s_ref[0, 0]


def _head(h0, h1r, ligidx, Wout0, C1, c1b, C2, c2b, Wout1):
    cat2, dxyz = pl.pallas_call(
        _head_body,
        out_shape=(
            jax.ShapeDtypeStruct((1, NOUT), jnp.float32),
            jax.ShapeDtypeStruct((1, 3), jnp.float32),
        ),
    )(h0, h1r, ligidx, Wout0, C1, c1b.reshape(1, -1), C2, c2b.reshape(1, -1),
      Wout1.reshape(1, 1))
    return cat2[0], dxyz


def kernel(x_bnd, x_res, x_atm_l1, pos_atm, pos_res, edge_feat_bnd,
           edge_feat_res, edge_feat_atm, r2a, ligidx, params,
           edge_index_bnd, edge_index_res, edge_index_atm):
    p = params
    sb, db = edge_index_bnd[0], edge_index_bnd[1]
    rlen_b = jnp.sqrt(jnp.sum((pos_atm[db] - pos_atm[sb]) ** 2, axis=-1) + 1e-12)
    h = jax.nn.elu(x_bnd @ p['W1_bnd'] + p['b1_bnd'])
    h = h @ p['W2_bnd'] + p['b2_bnd']
    for i in range(NL_BND):
        h = _scalar_layer(h, sb, db, edge_feat_bnd, rlen_b, p['Wq_bnd'][i],
                          p['Wk_bnd'][i], p['Wv_bnd'][i], p['We_bnd'][i],
                          p['Ws_bnd'][i], N_ATM)
    h_bnd = h @ p['Wout_bnd']

    sr, dr = edge_index_res[0], edge_index_res[1]
    rlen_r = jnp.sqrt(jnp.sum((pos_res[dr] - pos_res[sr]) ** 2, axis=-1) + 1e-12)
    h = jax.nn.elu(x_res @ p['W1_res'] + p['b1_res'])
    h = h @ p['W2_res'] + p['b2_res']
    for i in range(NL_RES):
        h = _scalar_layer(h, sr, dr, edge_feat_res, rlen_r, p['Wq_res'][i],
                          p['Wk_res'][i], p['Wv_res'][i], p['We_res'][i],
                          p['Ws_res'][i], N_RES)
    h_res = h @ p['Wout_res']

    h_resA = r2a @ h_res
    l0 = jnp.concatenate([h_bnd, h_resA], axis=1)
    l0 = jax.nn.elu(l0 @ p['W1_atm'] + p['b1_atm'])
    h0 = l0 @ p['W2_atm'] + p['b2_atm']
    h1 = x_atm_l1
    sa, da = edge_index_atm[0], edge_index_atm[1]
    rel_a = pos_atm[da] - pos_atm[sa]
    rlen_a = jnp.sqrt(jnp.sum(rel_a ** 2, axis=-1) + 1e-12)
    rhat = rel_a / (rlen_a[:, None] + 1e-6)
    radial = jnp.exp(-(rlen_a ** 2) / 100.0)[:, None]
    for i in range(NL_ATM):
        q = h0 @ p['Wq_atm'][i]
        kk = (h0[sa] @ p['Wk_atm'][i]) * (edge_feat_atm @ p['We_atm'][i])
        logits = jnp.sum(q[da] * kk, axis=-1) / jnp.sqrt(float(C))
        a = _seg_softmax(logits, da, N_ATM)
        v = (h0[sa] @ p['Wv_atm'][i]) * radial
        agg0 = jax.ops.segment_sum(a[:, None] * v, da, num_segments=N_ATM)
        h0n = agg0 + h0 @ p['Ws_atm'][i]
        nv = _safe_norm(h0n)
        h0n = h0n / (nv + 1e-6) * jax.nn.relu(nv)
        phi = h0[sa] @ p['Wdir_atm'][i]
        m1 = a[:, None, None] * (p['Wl1_atm'][i] * h1[sa] + phi[:, :, None] * rhat[:, None, :])
        h1 = jax.ops.segment_sum(m1, da, num_segments=N_ATM) + h1
        h0 = h0n
    cat, dxyz = _head(h0, h1[:, 0, :], ligidx, p['Wout0_atm'], p['C1'],
                      p['c1b'], p['C2'], p['c2b'], p['Wout1_atm'])
    return (cat, dxyz)


# speedup vs baseline: 1.3121x; 1.2976x over previous
import functools

import jax
import jax.numpy as jnp
from jax import lax
from jax.experimental import pallas as pl
from jax.experimental.pallas import tpu as pltpu
from jax.experimental.pallas import tpu_sc as plsc

N_ATM = 10000
N_RES = 1000
C = 32
NL_BND = 2
NL_RES = 4
NL_ATM = 4
NOUT = 11

_NC = 2
_NS = 16
_NW = _NC * _NS


def _sc_gather(tables, idxs, block):
    e = idxs[0].shape[0]
    epw = e // _NW
    nchunk = epw // block
    nt = len(tables)
    widths = [int(t.shape[1]) for t in tables]
    mesh = plsc.VectorSubcoreMesh(core_axis_name="c", subcore_axis_name="s",
                                  num_cores=_NC)
    out_ty = [jax.ShapeDtypeStruct((e, w), jnp.float32) for w in widths]
    scratch = ([pltpu.VMEM((block,), jnp.int32) for _ in range(nt)]
               + [pltpu.VMEM((block, w), jnp.float32) for w in widths]
               + [pltpu.SemaphoreType.DMA for _ in range(nt)])

    @functools.partial(
        pl.kernel, mesh=mesh, out_type=out_ty, scratch_types=scratch,
        compiler_params=pltpu.CompilerParams(use_tc_tiling_on_sc=False))
    def gk(*refs):
        tab = refs[:nt]
        idx = refs[nt:2 * nt]
        outs = refs[2 * nt:3 * nt]
        ibufs = refs[3 * nt:4 * nt]
        rbufs = refs[4 * nt:5 * nt]
        sems = refs[5 * nt:6 * nt]
        wid = lax.axis_index("s") * _NC + lax.axis_index("c")
        base = wid * epw

        def body(ci, carry):
            off = base + ci * block
            for j in range(nt):
                pltpu.sync_copy(idx[j].at[pl.ds(off, block)], ibufs[j])
            cps = [pltpu.async_copy(tab[j].at[ibufs[j]], rbufs[j], sems[j])
                   for j in range(nt)]
            for cp in cps:
                cp.wait()
            for j in range(nt):
                pltpu.sync_copy(rbufs[j], outs[j].at[pl.ds(off, block)])
            return carry

        lax.fori_loop(0, nchunk, body, 0)

    return gk(*tables, *idxs)


def _seg_softmax(logits, seg, n):
    m = jax.ops.segment_max(logits, seg, num_segments=n)
    m = jnp.where(jnp.isfinite(m), m, 0.0)
    e = jnp.exp(logits - m[seg])
    s = jax.ops.segment_sum(e, seg, num_segments=n)
    return e / (s[seg] + 1e-9)


def _safe_norm(x):
    return jnp.sqrt(jnp.sum(x * x, axis=-1, keepdims=True) + 1e-12)


def _scalar_layer(h0, src, dst, efeat, radial, Wq, Wk, Wv, We, Ws, n, blk):
    qn = h0 @ Wq
    kn = h0 @ Wk
    vn = h0 @ Wv
    qd, ks, vs = _sc_gather([qn, kn, vn], [dst, src, src], blk)
    kk = ks * (efeat @ We)
    logits = jnp.sum(qd * kk, axis=-1) / jnp.sqrt(float(h0.shape[1]))
    a = _seg_softmax(logits, dst, n)
    v = vs * radial
    agg = jax.ops.segment_sum(a[:, None] * v, dst, num_segments=n)
    out = agg + h0 @ Ws
    nv = _safe_norm(out)
    return out / (nv + 1e-6) * jax.nn.relu(nv)


def _head_body(h0_ref, h1_ref, lig_ref, w0_ref, c1_ref, c1b_ref, c2_ref,
               c2b_ref, w1s_ref, cat_ref, dxyz_ref):
    h0 = h0_ref[...]
    lig = lig_ref[...]
    h0o = jnp.dot(h0, w0_ref[...], preferred_element_type=jnp.float32)
    z = jax.nn.relu(jnp.dot(h0o, c1_ref[...],
                            preferred_element_type=jnp.float32) + c1b_ref[...])
    cat = jnp.dot(z, c2_ref[...], preferred_element_type=jnp.float32) + c2b_ref[...]
    cat_ref[...] = jnp.dot(lig, cat, preferred_element_type=jnp.float32)
    dxyz_ref[...] = jnp.dot(lig, h1_ref[...],
                            preferred_element_type=jnp.float32) * w1s_ref[0, 0]


def _head(h0, h1r, ligidx, Wout0, C1, c1b, C2, c2b, Wout1):
    cat2, dxyz = pl.pallas_call(
        _head_body,
        out_shape=(
            jax.ShapeDtypeStruct((1, NOUT), jnp.float32),
            jax.ShapeDtypeStruct((1, 3), jnp.float32),
        ),
    )(h0, h1r, ligidx, Wout0, C1, c1b.reshape(1, -1), C2, c2b.reshape(1, -1),
      Wout1.reshape(1, 1))
    return cat2[0], dxyz


def kernel(x_bnd, x_res, x_atm_l1, pos_atm, pos_res, edge_feat_bnd,
           edge_feat_res, edge_feat_atm, r2a, ligidx, params,
           edge_index_bnd, edge_index_res, edge_index_atm):
    p = params
    pos_atm16 = jnp.pad(pos_atm, ((0, 0), (0, 13)))
    pos_res16 = jnp.pad(pos_res, ((0, 0), (0, 13)))

    sb, db = edge_index_bnd[0], edge_index_bnd[1]
    pdb, psb = _sc_gather([pos_atm16, pos_atm16], [db, sb], 1000)
    r2_b = jnp.sum((pdb[:, :3] - psb[:, :3]) ** 2, axis=-1)
    radial_b = jnp.exp(-(r2_b + 1e-12) / 100.0)[:, None]
    h = jax.nn.elu(x_bnd @ p['W1_bnd'] + p['b1_bnd'])
    h = h @ p['W2_bnd'] + p['b2_bnd']
    for i in range(NL_BND):
        h = _scalar_layer(h, sb, db, edge_feat_bnd, radial_b, p['Wq_bnd'][i],
                          p['Wk_bnd'][i], p['Wv_bnd'][i], p['We_bnd'][i],
                          p['Ws_bnd'][i], N_ATM, 1000)
    h_bnd = h @ p['Wout_bnd']

    sr, dr = edge_index_res[0], edge_index_res[1]
    pdr, psr = _sc_gather([pos_res16, pos_res16], [dr, sr], 1000)
    r2_r = jnp.sum((pdr[:, :3] - psr[:, :3]) ** 2, axis=-1)
    radial_r = jnp.exp(-(r2_r + 1e-12) / 100.0)[:, None]
    h = jax.nn.elu(x_res @ p['W1_res'] + p['b1_res'])
    h = h @ p['W2_res'] + p['b2_res']
    for i in range(NL_RES):
        h = _scalar_layer(h, sr, dr, edge_feat_res, radial_r, p['Wq_res'][i],
                          p['Wk_res'][i], p['Wv_res'][i], p['We_res'][i],
                          p['Ws_res'][i], N_RES, 1000)
    h_res = h @ p['Wout_res']

    h_resA = r2a @ h_res
    l0 = jnp.concatenate([h_bnd, h_resA], axis=1)
    l0 = jax.nn.elu(l0 @ p['W1_atm'] + p['b1_atm'])
    h0 = l0 @ p['W2_atm'] + p['b2_atm']
    h1 = x_atm_l1[:, 0, :]

    sa, da = edge_index_atm[0], edge_index_atm[1]
    pda, psa = _sc_gather([pos_atm16, pos_atm16], [da, sa], 1000)
    rel_a = pda[:, :3] - psa[:, :3]
    r2_a = jnp.sum(rel_a ** 2, axis=-1)
    rlen_a = jnp.sqrt(r2_a + 1e-12)
    rhat = rel_a / (rlen_a[:, None] + 1e-6)
    radial = jnp.exp(-(r2_a + 1e-12) / 100.0)[:, None]
    for i in range(NL_ATM):
        qn = h0 @ p['Wq_atm'][i]
        kn = h0 @ p['Wk_atm'][i]
        vn = h0 @ p['Wv_atm'][i]
        phin = h0 @ p['Wdir_atm'][i]
        h1aug = jnp.concatenate(
            [h1, phin, jnp.zeros((N_ATM, 12), jnp.float32)], axis=1)
        qd, ks, vs, h1s = _sc_gather([qn, kn, vn, h1aug], [da, sa, sa, sa], 1000)
        kk = ks * (edge_feat_atm @ p['We_atm'][i])
        logits = jnp.sum(qd * kk, axis=-1) / jnp.sqrt(float(C))
        a = _seg_softmax(logits, da, N_ATM)
        v = vs * radial
        agg0 = jax.ops.segment_sum(a[:, None] * v, da, num_segments=N_ATM)
        h0n = agg0 + h0 @ p['Ws_atm'][i]
        nv = _safe_norm(h0n)
        h0n = h0n / (nv + 1e-6) * jax.nn.relu(nv)
        m1 = a[:, None] * (p['Wl1_atm'][i] * h1s[:, :3] + h1s[:, 3:4] * rhat)
        h1 = jax.ops.segment_sum(m1, da, num_segments=N_ATM) + h1
        h0 = h0n
    cat, dxyz = _head(h0, h1, ligidx, p['Wout0_atm'], p['C1'],
                      p['c1b'], p['C2'], p['c2b'], p['Wout1_atm'])
    return (cat, dxyz)


# speedup vs baseline: 1.4484x; 1.1039x over previous
import functools

import jax
import jax.numpy as jnp
from jax import lax
from jax.experimental import pallas as pl
from jax.experimental.pallas import tpu as pltpu
from jax.experimental.pallas import tpu_sc as plsc

N_ATM = 10000
N_RES = 1000
C = 32
NL_BND = 2
NL_RES = 4
NL_ATM = 4
NOUT = 11

_NC = 2
_NS = 16
_NW = _NC * _NS


def _sc_gather(tables, idxs, block):
    e = idxs[0].shape[0]
    epw = e // _NW
    nchunk = epw // block
    nt = len(tables)
    widths = [int(t.shape[1]) for t in tables]
    mesh = plsc.VectorSubcoreMesh(core_axis_name="c", subcore_axis_name="s",
                                  num_cores=_NC)
    out_ty = [jax.ShapeDtypeStruct((e, w), jnp.float32) for w in widths]
    scratch = ([pltpu.VMEM((block,), jnp.int32) for _ in range(nt)]
               + [pltpu.VMEM((block, w), jnp.float32) for w in widths]
               + [pltpu.SemaphoreType.DMA for _ in range(nt)])

    @functools.partial(
        pl.kernel, mesh=mesh, out_type=out_ty, scratch_types=scratch,
        compiler_params=pltpu.CompilerParams(use_tc_tiling_on_sc=False))
    def gk(*refs):
        tab = refs[:nt]
        idx = refs[nt:2 * nt]
        outs = refs[2 * nt:3 * nt]
        ibufs = refs[3 * nt:4 * nt]
        rbufs = refs[4 * nt:5 * nt]
        sems = refs[5 * nt:6 * nt]
        wid = lax.axis_index("s") * _NC + lax.axis_index("c")
        base = wid * epw

        def body(ci, carry):
            off = base + ci * block
            for j in range(nt):
                pltpu.sync_copy(idx[j].at[pl.ds(off, block)], ibufs[j])
            cps = [pltpu.async_copy(tab[j].at[ibufs[j]], rbufs[j], sems[j])
                   for j in range(nt)]
            for cp in cps:
                cp.wait()
            for j in range(nt):
                pltpu.sync_copy(rbufs[j], outs[j].at[pl.ds(off, block)])
            return carry

        lax.fori_loop(0, nchunk, body, 0)

    return gk(*tables, *idxs)


def _sc_scatter_add(rows, dst, n, block):
    e, w = rows.shape
    n_pad = ((n + 15) // 16) * 16
    rpt = n_pad // _NS
    epw = e // _NW
    nchunk = epw // block
    mesh = plsc.VectorSubcoreMesh(core_axis_name="c", subcore_axis_name="s",
                                  num_cores=_NC)

    @functools.partial(
        pl.kernel, mesh=mesh,
        out_type=jax.ShapeDtypeStruct((_NC, n_pad, w), jnp.float32),
        scratch_types=[
            pltpu.VMEM((block,), jnp.int32),
            pltpu.VMEM((block, w), jnp.float32),
            pltpu.VMEM((rpt, w), jnp.float32),
            pltpu.VMEM_SHARED((n_pad, w), jnp.float32),
        ],
        compiler_params=pltpu.CompilerParams(use_tc_tiling_on_sc=False))
    def sk(rows_h, dst_h, out_h, ibuf, rbuf, obuf, acc_sh):
        cid = lax.axis_index("c")
        sid = lax.axis_index("s")
        wid = sid * _NC + cid
        base = wid * epw

        def zrow(r, carry):
            obuf[r, 0:16] = jnp.zeros((16,), jnp.float32)
            obuf[r, 16:32] = jnp.zeros((16,), jnp.float32)
            return carry

        if w == 16:
            def zrow(r, carry):
                obuf[r, 0:16] = jnp.zeros((16,), jnp.float32)
                return carry

        lax.fori_loop(0, rpt, zrow, 0)
        pltpu.sync_copy(obuf, acc_sh.at[pl.ds(sid * rpt, rpt)])
        plsc.subcore_barrier()

        def body(ci, carry):
            off = base + ci * block
            pltpu.sync_copy(dst_h.at[pl.ds(off, block)], ibuf)
            pltpu.sync_copy(rows_h.at[pl.ds(off, block)], rbuf)
            pltpu.sync_copy(rbuf, acc_sh.at[ibuf], add=True)
            return carry

        lax.fori_loop(0, nchunk, body, 0)
        plsc.subcore_barrier()
        pltpu.sync_copy(acc_sh.at[pl.ds(sid * rpt, rpt)], obuf)
        pltpu.sync_copy(obuf, out_h.at[cid, pl.ds(sid * rpt, rpt)])

    out2 = sk(rows, dst)
    return (out2[0] + out2[1])[:n]


def _seg_softmax(logits, seg, n):
    m = jax.ops.segment_max(logits, seg, num_segments=n)
    m = jnp.where(jnp.isfinite(m), m, 0.0)
    e = jnp.exp(logits - m[seg])
    s = jax.ops.segment_sum(e, seg, num_segments=n)
    return e / (s[seg] + 1e-9)


def _safe_norm(x):
    return jnp.sqrt(jnp.sum(x * x, axis=-1, keepdims=True) + 1e-12)


def _scalar_layer(h0, src, dst, efeat, radial, Wq, Wk, Wv, We, Ws, n, blk):
    qn = h0 @ Wq
    kn = h0 @ Wk
    vn = h0 @ Wv
    qd, ks, vs = _sc_gather([qn, kn, vn], [dst, src, src], blk)
    kk = ks * (efeat @ We)
    logits = jnp.sum(qd * kk, axis=-1) / jnp.sqrt(float(h0.shape[1]))
    a = _seg_softmax(logits, dst, n)
    v = vs * radial
    agg = _sc_scatter_add(a[:, None] * v, dst, n, blk)
    out = agg + h0 @ Ws
    nv = _safe_norm(out)
    return out / (nv + 1e-6) * jax.nn.relu(nv)


def _head_body(h0_ref, h1_ref, lig_ref, w0_ref, c1_ref, c1b_ref, c2_ref,
               c2b_ref, w1s_ref, cat_ref, dxyz_ref):
    h0 = h0_ref[...]
    lig = lig_ref[...]
    h0o = jnp.dot(h0, w0_ref[...], preferred_element_type=jnp.float32)
    z = jax.nn.relu(jnp.dot(h0o, c1_ref[...],
                            preferred_element_type=jnp.float32) + c1b_ref[...])
    cat = jnp.dot(z, c2_ref[...], preferred_element_type=jnp.float32) + c2b_ref[...]
    cat_ref[...] = jnp.dot(lig, cat, preferred_element_type=jnp.float32)
    dxyz_ref[...] = jnp.dot(lig, h1_ref[...],
                            preferred_element_type=jnp.float32) * w1s_ref[0, 0]


def _head(h0, h1r, ligidx, Wout0, C1, c1b, C2, c2b, Wout1):
    cat2, dxyz = pl.pallas_call(
        _head_body,
        out_shape=(
            jax.ShapeDtypeStruct((1, NOUT), jnp.float32),
            jax.ShapeDtypeStruct((1, 3), jnp.float32),
        ),
    )(h0, h1r, ligidx, Wout0, C1, c1b.reshape(1, -1), C2, c2b.reshape(1, -1),
      Wout1.reshape(1, 1))
    return cat2[0], dxyz


def kernel(x_bnd, x_res, x_atm_l1, pos_atm, pos_res, edge_feat_bnd,
           edge_feat_res, edge_feat_atm, r2a, ligidx, params,
           edge_index_bnd, edge_index_res, edge_index_atm):
    p = params
    pos_atm16 = jnp.pad(pos_atm, ((0, 0), (0, 13)))
    pos_res16 = jnp.pad(pos_res, ((0, 0), (0, 13)))

    sb, db = edge_index_bnd[0], edge_index_bnd[1]
    pdb, psb = _sc_gather([pos_atm16, pos_atm16], [db, sb], 1000)
    r2_b = jnp.sum((pdb[:, :3] - psb[:, :3]) ** 2, axis=-1)
    radial_b = jnp.exp(-(r2_b + 1e-12) / 100.0)[:, None]
    h = jax.nn.elu(x_bnd @ p['W1_bnd'] + p['b1_bnd'])
    h = h @ p['W2_bnd'] + p['b2_bnd']
    for i in range(NL_BND):
        h = _scalar_layer(h, sb, db, edge_feat_bnd, radial_b, p['Wq_bnd'][i],
                          p['Wk_bnd'][i], p['Wv_bnd'][i], p['We_bnd'][i],
                          p['Ws_bnd'][i], N_ATM, 1000)
    h_bnd = h @ p['Wout_bnd']

    sr, dr = edge_index_res[0], edge_index_res[1]
    pdr, psr = _sc_gather([pos_res16, pos_res16], [dr, sr], 1000)
    r2_r = jnp.sum((pdr[:, :3] - psr[:, :3]) ** 2, axis=-1)
    radial_r = jnp.exp(-(r2_r + 1e-12) / 100.0)[:, None]
    h = jax.nn.elu(x_res @ p['W1_res'] + p['b1_res'])
    h = h @ p['W2_res'] + p['b2_res']
    for i in range(NL_RES):
        h = _scalar_layer(h, sr, dr, edge_feat_res, radial_r, p['Wq_res'][i],
                          p['Wk_res'][i], p['Wv_res'][i], p['We_res'][i],
                          p['Ws_res'][i], N_RES, 1000)
    h_res = h @ p['Wout_res']

    h_resA = r2a @ h_res
    l0 = jnp.concatenate([h_bnd, h_resA], axis=1)
    l0 = jax.nn.elu(l0 @ p['W1_atm'] + p['b1_atm'])
    h0 = l0 @ p['W2_atm'] + p['b2_atm']
    h1 = x_atm_l1[:, 0, :]

    sa, da = edge_index_atm[0], edge_index_atm[1]
    pda, psa = _sc_gather([pos_atm16, pos_atm16], [da, sa], 1000)
    rel_a = pda[:, :3] - psa[:, :3]
    r2_a = jnp.sum(rel_a ** 2, axis=-1)
    rlen_a = jnp.sqrt(r2_a + 1e-12)
    rhat = rel_a / (rlen_a[:, None] + 1e-6)
    radial = jnp.exp(-(r2_a + 1e-12) / 100.0)[:, None]
    for i in range(NL_ATM):
        qn = h0 @ p['Wq_atm'][i]
        kn = h0 @ p['Wk_atm'][i]
        vn = h0 @ p['Wv_atm'][i]
        phin = h0 @ p['Wdir_atm'][i]
        h1aug = jnp.concatenate(
            [h1, phin, jnp.zeros((N_ATM, 12), jnp.float32)], axis=1)
        qd, ks, vs, h1s = _sc_gather([qn, kn, vn, h1aug], [da, sa, sa, sa], 1000)
        kk = ks * (edge_feat_atm @ p['We_atm'][i])
        logits = jnp.sum(qd * kk, axis=-1) / jnp.sqrt(float(C))
        a = _seg_softmax(logits, da, N_ATM)
        v = vs * radial
        agg0 = _sc_scatter_add(a[:, None] * v, da, N_ATM, 1000)
        h0n = agg0 + h0 @ p['Ws_atm'][i]
        nv = _safe_norm(h0n)
        h0n = h0n / (nv + 1e-6) * jax.nn.relu(nv)
        m1 = a[:, None] * (p['Wl1_atm'][i] * h1s[:, :3] + h1s[:, 3:4] * rhat)
        m1p = jnp.pad(m1, ((0, 0), (0, 13)))
        h1 = _sc_scatter_add(m1p, da, N_ATM, 1000)[:, :3] + h1
        h0 = h0n
    cat, dxyz = _head(h0, h1, ligidx, p['Wout0_atm'], p['C1'],
                      p['c1b'], p['C2'], p['c2b'], p['Wout1_atm'])
    return (cat, dxyz)
